# Initial kernel scaffold; baseline (speedup 1.0000x reference)
import jax
import jax.numpy as jnp
from jax.experimental import pallas as pl


def kernel(batch_x, batch_edge_index, batch_edge_type, comp, bases, root, bias, W_global, b_global, W_sense, b_sense):
    raise NotImplementedError("write your pallas kernel here")


# speedup vs baseline: 97.4801x; 97.4801x over previous
import functools

import jax
import jax.numpy as jnp
from jax import lax
from jax.experimental import pallas as pl
from jax.experimental.pallas import tpu as pltpu
from jax.experimental.pallas import tpu_sc as plsc

_L = 16
_ACC_ROWS = 8


def _sc_scan_body(chunk, nblk, n_rel, num_cores,
                  x_hbm, src_hbm, dst_hbm, typ_hbm, part_hbm, cnt_hbm,
                  src_v, dst_v, typ_v, idx_v, tdx_v, rows_v, acc_v, cnt_v):
    sid = lax.axis_index("s")
    wid = sid * num_cores + lax.axis_index("c")
    base = wid * chunk

    pltpu.sync_copy(src_hbm.at[pl.ds(base, chunk)], src_v.at[pl.ds(0, chunk)])
    pltpu.sync_copy(dst_hbm.at[pl.ds(base, chunk)], dst_v.at[pl.ds(0, chunk)])
    pltpu.sync_copy(typ_hbm.at[pl.ds(base, chunk)], typ_v.at[pl.ds(0, chunk)])

    zf = jnp.zeros((_L,), jnp.float32)
    for row in range(_ACC_ROWS):
        cnt_v[row, :] = zf
        for j in range(rows_v.shape[1] // _L):
            acc_v[row, pl.ds(j * _L, _L)] = zf

    def _mask_at(off):
        dstv = dst_v[pl.ds(off, _L)]
        lanes = lax.iota(jnp.int32, _L)
        return (dstv == 0) & ((off + lanes) < chunk)

    def step(i, carry):
        off = i * _L
        nhit = jnp.sum(_mask_at(off).astype(jnp.int32))

        @pl.when(nhit > 0)
        def _hit():
            mask = _mask_at(off)
            srcv = src_v[pl.ds(off, _L)]
            typv = typ_v[pl.ds(off, _L)]
            idx_v[...] = jnp.where(mask, srcv, 0)
            tdx = jnp.where(mask, typv, _ACC_ROWS - 1)
            tdx_v[...] = tdx
            pltpu.sync_copy(x_hbm.at[idx_v], rows_v)
            nchunks = rows_v.shape[1] // _L
            for lane in range(_L):
                t_l = tdx[lane]
                for j in range(nchunks):
                    sl = pl.ds(j * _L, _L)
                    plsc.addupdate(acc_v.at[t_l, sl], rows_v[lane, sl])
            ones = jnp.where(mask, 1.0, 0.0).astype(jnp.float32)
            zero = jnp.zeros((_L,), jnp.float32)
            for r in range(n_rel):
                cnt_v[r, :] = cnt_v[r, :] + jnp.where(typv == r, ones, zero)

        return carry

    lax.fori_loop(0, nblk, step, 0)

    pltpu.sync_copy(acc_v, part_hbm.at[wid])
    pltpu.sync_copy(cnt_v, cnt_hbm.at[wid])


def _tc_tail_body(n_rel, part_ref, cnt_ref, comp_ref, bases_ref, root_ref,
                  bias_ref, x0_ref, wg_ref, bg_ref, ws_ref, bs_ref,
                  outg_ref, outs_ref):
    s = jnp.sum(part_ref[...], axis=0)
    cnt = jnp.sum(cnt_ref[...], axis=(0, 2))
    m = s / jnp.maximum(cnt, 1.0)[:, None]
    v = lax.dot_general(comp_ref[...], m[0:n_rel, :],
                        (((0,), (0,)), ((), ())),
                        preferred_element_type=jnp.float32)
    out0 = jnp.dot(x0_ref[...], root_ref[...],
                   preferred_element_type=jnp.float32) + bias_ref[...]
    for r in range(n_rel):
        out0 = out0 + jnp.dot(v[r:r + 1, :], bases_ref[r],
                              preferred_element_type=jnp.float32)
    h = jnp.maximum(out0, 0.0)

    def head(w_ref, b_ref, o_ref):
        lg = lax.dot_general(h, w_ref[...], (((1,), (1,)), ((), ())),
                             preferred_element_type=jnp.float32) + b_ref[...]
        mx = jnp.max(lg, axis=1, keepdims=True)
        lse = jnp.log(jnp.sum(jnp.exp(lg - mx), axis=1, keepdims=True))
        o_ref[...] = lg - mx - lse

    head(wg_ref, bg_ref, outg_ref)
    head(ws_ref, bs_ref, outs_ref)


def kernel(batch_x, batch_edge_index, batch_edge_type, comp, bases, root,
           bias, W_global, b_global, W_sense, b_sense):
    n, d = batch_x.shape
    e = batch_edge_type.shape[0]
    n_rel = comp.shape[0]
    g = W_global.shape[0]
    s_cnt = W_sense.shape[0]

    info = plsc.get_sparse_core_info()
    nw = info.num_cores * info.num_subcores
    chunk = e // nw
    nblk = (chunk + _L - 1) // _L
    padded = nblk * _L

    src = batch_edge_index[0]
    dst = batch_edge_index[1]

    sc_fn = pl.kernel(
        functools.partial(_sc_scan_body, chunk, nblk, n_rel, info.num_cores),
        out_type=(jax.ShapeDtypeStruct((nw, _ACC_ROWS, d), jnp.float32),
                  jax.ShapeDtypeStruct((nw, _ACC_ROWS, _L), jnp.float32)),
        mesh=plsc.VectorSubcoreMesh(core_axis_name="c", subcore_axis_name="s"),
        compiler_params=pltpu.CompilerParams(needs_layout_passes=False),
        scratch_types=[
            pltpu.VMEM((padded,), jnp.int32),
            pltpu.VMEM((padded,), jnp.int32),
            pltpu.VMEM((padded,), jnp.int32),
            pltpu.VMEM((_L,), jnp.int32),
            pltpu.VMEM((_L,), jnp.int32),
            pltpu.VMEM((_L, d), jnp.float32),
            pltpu.VMEM((_ACC_ROWS, d), jnp.float32),
            pltpu.VMEM((_ACC_ROWS, _L), jnp.float32),
        ],
    )
    part, cnts = sc_fn(batch_x, src, dst, batch_edge_type)

    x0 = lax.slice(batch_x, (0, 0), (1, d))
    outg, outs = pl.pallas_call(
        functools.partial(_tc_tail_body, n_rel),
        out_shape=(jax.ShapeDtypeStruct((1, g), jnp.float32),
                   jax.ShapeDtypeStruct((1, s_cnt), jnp.float32)),
    )(part, cnts, comp, bases, root, bias.reshape(1, d), x0,
      W_global, b_global.reshape(1, g), W_sense, b_sense.reshape(1, s_cnt))

    return (outg.reshape(g), outs.reshape(s_cnt))


# speedup vs baseline: 103.0084x; 1.0567x over previous
import functools

import jax
import jax.numpy as jnp
from jax import lax
from jax.experimental import pallas as pl
from jax.experimental.pallas import tpu as pltpu
from jax.experimental.pallas import tpu_sc as plsc

_L = 16
_ACC_ROWS = 8
_CNT_ROW = 5
_SUPER = 16


def _sc_scan_body(chunk, nsuper, n_rel, num_cores,
                  x_hbm, src_hbm, dst_hbm, typ_hbm, part_hbm, cnt_hbm,
                  src_v, dst_v, typ_v, idx_v, rows_v, acc_v, cnt_v, sem):
    sid = lax.axis_index("s")
    wid = sid * num_cores + lax.axis_index("c")
    base = wid * chunk

    cp_src = pltpu.async_copy(src_hbm.at[pl.ds(base, chunk)],
                              src_v.at[pl.ds(0, chunk)], sem)
    cp_dst = pltpu.async_copy(dst_hbm.at[pl.ds(base, chunk)],
                              dst_v.at[pl.ds(0, chunk)], sem)
    cp_typ = pltpu.async_copy(typ_hbm.at[pl.ds(base, chunk)],
                              typ_v.at[pl.ds(0, chunk)], sem)

    zf = jnp.zeros((_L,), jnp.float32)
    for row in range(_ACC_ROWS):
        cnt_v[row, :] = zf
        for j in range(rows_v.shape[1] // _L):
            acc_v[row, pl.ds(j * _L, _L)] = zf

    cp_src.wait()
    cp_dst.wait()
    cp_typ.wait()

    ones_i = jnp.ones((_L,), jnp.int32)
    lanes = lax.iota(jnp.int32, _L)
    tail = (chunk // _L) * _L
    rem = chunk - tail
    if rem:
        blk = dst_v[pl.ds(tail, _L)]
        dst_v[pl.ds(tail, _L)] = jnp.where(lanes < rem, blk, ones_i)
    for b in range((chunk + _L - 1) // _L, nsuper * _SUPER):
        dst_v[pl.ds(b * _L, _L)] = ones_i

    def fine_step(b, carry):
        off = b * _L
        dstv = dst_v[pl.ds(off, _L)]
        mask = dstv == 0
        nhit = jnp.sum(mask.astype(jnp.int32))

        @pl.when(nhit > 0)
        def _hit():
            dstv2 = dst_v[pl.ds(off, _L)]
            mask2 = dstv2 == 0
            srcv = src_v[pl.ds(off, _L)]
            typv = typ_v[pl.ds(off, _L)]
            idx_v[...] = jnp.where(mask2, srcv, 0)
            tdx = jnp.where(mask2, typv, _ACC_ROWS - 1)
            pltpu.sync_copy(x_hbm.at[idx_v], rows_v)
            nchunks = rows_v.shape[1] // _L
            for lane in range(_L):
                t_l = tdx[lane]
                for j in range(nchunks):
                    sl = pl.ds(j * _L, _L)
                    plsc.addupdate(acc_v.at[t_l, sl], rows_v[lane, sl])
            ones_f = jnp.where(mask2, 1.0, 0.0).astype(jnp.float32)
            zero_f = jnp.zeros((_L,), jnp.float32)
            for r in range(n_rel):
                cnt_v[r, :] = cnt_v[r, :] + jnp.where(typv == r, ones_f, zero_f)

        return carry

    def super_step(s, carry):
        soff = s * _SUPER * _L
        mv = dst_v[pl.ds(soff, _L)]
        for b in range(1, _SUPER):
            mv = jnp.minimum(mv, dst_v[pl.ds(soff + b * _L, _L)])
        nz = jnp.sum(jnp.where(mv == 0, 1, 0))

        @pl.when(nz > 0)
        def _scan_fine():
            lax.fori_loop(s * _SUPER, (s + 1) * _SUPER, fine_step, 0)

        return carry

    lax.fori_loop(0, nsuper, super_step, 0)

    pltpu.sync_copy(acc_v, part_hbm.at[wid])
    pltpu.sync_copy(cnt_v, cnt_hbm.at[wid])


def _tc_tail_body(n_rel, part_ref, cnt_ref, comp_ref, bases_ref, root_ref,
                  bias_ref, x0_ref, wg_ref, bg_ref, ws_ref, bs_ref,
                  outg_ref, outs_ref):
    s = jnp.sum(part_ref[...], axis=0)
    cnt = jnp.sum(cnt_ref[...], axis=(0, 2))
    m = s[0:n_rel, :] / jnp.maximum(cnt[0:n_rel], 1.0)[:, None]
    v = lax.dot_general(comp_ref[...], m,
                        (((0,), (0,)), ((), ())),
                        preferred_element_type=jnp.float32)
    out0 = jnp.dot(x0_ref[...], root_ref[...],
                   preferred_element_type=jnp.float32) + bias_ref[...]
    for r in range(n_rel):
        out0 = out0 + jnp.dot(v[r:r + 1, :], bases_ref[r],
                              preferred_element_type=jnp.float32)
    h = jnp.maximum(out0, 0.0)

    def head(w_ref, b_ref, o_ref):
        lg = lax.dot_general(h, w_ref[...], (((1,), (1,)), ((), ())),
                             preferred_element_type=jnp.float32) + b_ref[...]
        mx = jnp.max(lg, axis=1, keepdims=True)
        lse = jnp.log(jnp.sum(jnp.exp(lg - mx), axis=1, keepdims=True))
        o_ref[...] = lg - mx - lse

    head(wg_ref, bg_ref, outg_ref)
    head(ws_ref, bs_ref, outs_ref)


def kernel(batch_x, batch_edge_index, batch_edge_type, comp, bases, root,
           bias, W_global, b_global, W_sense, b_sense):
    n, d = batch_x.shape
    e = batch_edge_type.shape[0]
    n_rel = comp.shape[0]
    g = W_global.shape[0]
    s_cnt = W_sense.shape[0]

    info = plsc.get_sparse_core_info()
    nw = info.num_cores * info.num_subcores
    chunk = e // nw
    nblk = (chunk + _L - 1) // _L
    nsuper = (nblk + _SUPER - 1) // _SUPER
    padded = nsuper * _SUPER * _L

    sc_fn = pl.kernel(
        functools.partial(_sc_scan_body, chunk, nsuper, n_rel,
                          info.num_cores),
        out_type=(jax.ShapeDtypeStruct((nw, _ACC_ROWS, d), jnp.float32),
                  jax.ShapeDtypeStruct((nw, _ACC_ROWS, _L), jnp.float32)),
        mesh=plsc.VectorSubcoreMesh(core_axis_name="c", subcore_axis_name="s"),
        compiler_params=pltpu.CompilerParams(needs_layout_passes=False),
        scratch_types=[
            pltpu.VMEM((padded,), jnp.int32),
            pltpu.VMEM((padded,), jnp.int32),
            pltpu.VMEM((padded,), jnp.int32),
            pltpu.VMEM((_L,), jnp.int32),
            pltpu.VMEM((_L, d), jnp.float32),
            pltpu.VMEM((_ACC_ROWS, d), jnp.float32),
            pltpu.VMEM((_ACC_ROWS, _L), jnp.float32),
            pltpu.SemaphoreType.DMA,
        ],
    )
    src = batch_edge_index[0]
    dst = batch_edge_index[1]
    part, cnts = sc_fn(batch_x, src, dst, batch_edge_type)

    x0 = lax.slice(batch_x, (0, 0), (1, d))
    outg, outs = pl.pallas_call(
        functools.partial(_tc_tail_body, n_rel),
        out_shape=(jax.ShapeDtypeStruct((1, g), jnp.float32),
                   jax.ShapeDtypeStruct((1, s_cnt), jnp.float32)),
    )(part, cnts, comp, bases, root, bias.reshape(1, d), x0,
      W_global, b_global.reshape(1, g), W_sense, b_sense.reshape(1, s_cnt))

    return (outg.reshape(g), outs.reshape(s_cnt))
